# baseline (speedup 1.0000x reference)
import jax
import jax.numpy as jnp
from jax import lax
from jax.experimental import pallas as pl
from jax.experimental.pallas import tpu as pltpu
from jax.experimental.pallas import tpu_sc as plsc

VOCAB = 1000000
DIM = 32
RANK = 8
SCALE = 1.0

NUM_CORES = 2
NUM_SUBCORES = 16
NW = NUM_CORES * NUM_SUBCORES
N_TOK = 16384 * 50
TOK_PER_W = N_TOK // NW
CB = 16
C = CB * 50
N_CHUNK = TOK_PER_W // C
ROWS_PER_W = 16384 // NW


def _body(weight_hbm, idx_hbm, lora_a_hbm, lora_b_hbm, out_hbm,
          idx_v0, idx_v1, sw_v0, sw_v1, sa_v0, sa_v1,
          w_v0, w_v1, a_v0, a_v1, lb_v,
          sem_w0, sem_w1, sem_a0, sem_a1, sem_o0, sem_o1):
    wid = lax.axis_index("s") * NUM_CORES + lax.axis_index("c")
    base = wid * TOK_PER_W
    rbase = wid * ROWS_PER_W

    pltpu.sync_copy(lora_b_hbm, lb_v)

    lane = lax.iota(jnp.int32, 16)
    row_step = lax.select(lane >= 8, jnp.ones((16,), jnp.int32),
                          jnp.zeros((16,), jnp.int32))
    col_idx = lax.rem(lane, jnp.full((16,), RANK, jnp.int32))

    bufs = ((idx_v0, sw_v0, sa_v0, w_v0, a_v0, sem_w0, sem_a0, sem_o0),
            (idx_v1, sw_v1, sa_v1, w_v1, a_v1, sem_w1, sem_a1, sem_o1))

    def start_gathers(ci, b):
        idx_v, sw_v, sa_v, w_v, a_v, sem_w, sem_a, _ = bufs[b]
        tok = pl.multiple_of(base + ci * C, C)
        pltpu.sync_copy(idx_hbm.at[pl.ds(tok, C)], idx_v)

        def perm(g):
            off = pl.multiple_of(g * 16, 16)
            v = idx_v[pl.ds(off, 16)]
            a7 = v & 127
            rw = v & 511
            sw_v[pl.ds(off, 16)] = (v - rw) + (a7 << 2) + (rw >> 7)
            ra = v & 2047
            sa_v[pl.ds(off, 16)] = (v - ra) + (a7 << 4) + (ra >> 7)

        plsc.parallel_loop(0, C // 16, 1, unroll=4, carry=None)(perm)
        pltpu.make_async_copy(weight_hbm.at[sw_v], w_v, sem_w).start()
        pltpu.make_async_copy(lora_a_hbm.at[sa_v], a_v, sem_a).start()

    def wait_gathers(b):
        idx_v, sw_v, sa_v, w_v, a_v, sem_w, sem_a, _ = bufs[b]
        pltpu.make_async_copy(weight_hbm.at[sw_v], w_v, sem_w).wait()
        pltpu.make_async_copy(lora_a_hbm.at[sa_v], a_v, sem_a).wait()

    def start_write(ci, b):
        w_v, sem_o = bufs[b][3], bufs[b][7]
        row = pl.multiple_of(rbase + ci * CB, CB)
        for j in range(CB):
            pltpu.make_async_copy(
                w_v.at[pl.ds(j * 50, 50)], out_hbm.at[row + j], sem_o).start()

    def wait_write(b):
        w_v, sem_o = bufs[b][3], bufs[b][7]
        for j in range(CB):
            pltpu.make_async_copy(
                w_v.at[pl.ds(0, 50)], out_hbm.at[0], sem_o).wait()

    def compute(b):
        w_v, a_v = bufs[b][3], bufs[b][4]

        def pair_body(p):
            t0 = p * 2
            av = plsc.load_gather(a_v, [t0 + row_step, col_idx])
            acc00 = w_v[t0, pl.ds(0, 16)]
            acc01 = w_v[t0, pl.ds(16, 16)]
            acc10 = w_v[t0 + 1, pl.ds(0, 16)]
            acc11 = w_v[t0 + 1, pl.ds(16, 16)]
            for r in range(RANK):
                b0 = lb_v[r, pl.ds(0, 16)]
                b1 = lb_v[r, pl.ds(16, 16)]
                s0 = av[r] * SCALE
                s1 = av[r + RANK] * SCALE
                acc00 = acc00 + s0 * b0
                acc01 = acc01 + s0 * b1
                acc10 = acc10 + s1 * b0
                acc11 = acc11 + s1 * b1
            w_v[t0, pl.ds(0, 16)] = acc00
            w_v[t0, pl.ds(16, 16)] = acc01
            w_v[t0 + 1, pl.ds(0, 16)] = acc10
            w_v[t0 + 1, pl.ds(16, 16)] = acc11

        plsc.parallel_loop(0, C // 2, 1, unroll=4, carry=None)(pair_body)

    start_gathers(0, 0)

    def outer(cj, _):
        for b in (0, 1):
            ci = cj * 2 + b
            wait_gathers(b)
            nb = 1 - b
            @pl.when(ci >= 1)
            def _():
                wait_write(nb)

            @pl.when(ci + 1 < N_CHUNK)
            def _():
                start_gathers(ci + 1, nb)

            compute(b)
            start_write(ci, b)
        return 0

    lax.fori_loop(0, N_CHUNK // 2, outer, 0)
    wait_write(1)


def _detile_body(x_ref, y_ref):
    x = x_ref[...]
    rows, vb = x.shape
    npc = vb // 128
    pieces = [jnp.transpose(x[:, j * 128:(j + 1) * 128]) for j in range(npc)]
    y_ref[...] = jnp.concatenate(pieces, axis=1)


def _detile(xt, vb):
    import math
    rows, v = xt.shape
    grid = math.ceil(v / vb)
    return pl.pallas_call(
        _detile_body,
        grid=(grid,),
        in_specs=[pl.BlockSpec((rows, vb), lambda i: (0, i))],
        out_specs=pl.BlockSpec((128, 128), lambda i: (i, 0)),
        out_shape=jax.ShapeDtypeStruct((grid * 128, 128), jnp.float32),
    )(xt)


@jax.jit
def _lora_embed(weight, idx, lora_a, lora_b):
    mesh = plsc.VectorSubcoreMesh(core_axis_name="c", subcore_axis_name="s")
    fn = pl.kernel(
        _body,
        out_type=jax.ShapeDtypeStruct((16384, 50, DIM), jnp.float32),
        mesh=mesh,
        compiler_params=pltpu.CompilerParams(
            needs_layout_passes=False, use_tc_tiling_on_sc=False),
        scratch_types=[
            pltpu.VMEM((C,), jnp.int32),
            pltpu.VMEM((C,), jnp.int32),
            pltpu.VMEM((C,), jnp.int32),
            pltpu.VMEM((C,), jnp.int32),
            pltpu.VMEM((C,), jnp.int32),
            pltpu.VMEM((C,), jnp.int32),
            pltpu.VMEM((C, DIM), jnp.float32),
            pltpu.VMEM((C, DIM), jnp.float32),
            pltpu.VMEM((C, RANK), jnp.float32),
            pltpu.VMEM((C, RANK), jnp.float32),
            pltpu.VMEM((RANK, DIM), jnp.float32),
            pltpu.SemaphoreType.DMA,
            pltpu.SemaphoreType.DMA,
            pltpu.SemaphoreType.DMA,
            pltpu.SemaphoreType.DMA,
            pltpu.SemaphoreType.DMA,
            pltpu.SemaphoreType.DMA,
        ],
    )
    return fn(weight, idx, lora_a, lora_b)


def kernel(x, weight, lora_A, lora_B):
    idx = x.reshape(-1).astype(jnp.int32)
    wperm = _detile(weight.T, 512)
    w_rm = wperm.reshape(-1, DIM)
    aperm = _detile(lora_A.T, 2048)
    a_rm = aperm.reshape(-1, RANK)
    return _lora_embed(w_rm, idx, a_rm, lora_B)


# speedup vs baseline: 1.7741x; 1.7741x over previous
import jax
import jax.numpy as jnp
from jax import lax
from jax.experimental import pallas as pl
from jax.experimental.pallas import tpu as pltpu
from jax.experimental.pallas import tpu_sc as plsc

VOCAB = 1000000
DIM = 32
RANK = 8
SCALE = 1.0

NUM_CORES = 2
NUM_SUBCORES = 16
NW = NUM_CORES * NUM_SUBCORES
N_TOK = 16384 * 50
TOK_PER_W = N_TOK // NW
CB = 16
C = CB * 50
N_CHUNK = TOK_PER_W // C
ROWS_PER_W = 16384 // NW


def _body(weight_hbm, idx_hbm, lora_a_hbm, lora_b_hbm, out_hbm,
          idx_v0, idx_v1, sw_v0, sw_v1, sa_v0, sa_v1,
          w_v0, w_v1, a_v0, a_v1, lb_v,
          sem_w0, sem_w1, sem_a0, sem_a1, sem_o0, sem_o1):
    wid = lax.axis_index("s") * NUM_CORES + lax.axis_index("c")
    base = wid * TOK_PER_W
    rbase = wid * ROWS_PER_W

    pltpu.sync_copy(lora_b_hbm, lb_v)

    lane = lax.iota(jnp.int32, 16)
    row_step = lax.select(lane >= 8, jnp.ones((16,), jnp.int32),
                          jnp.zeros((16,), jnp.int32))
    col_idx = lax.rem(lane, jnp.full((16,), RANK, jnp.int32))

    bufs = ((idx_v0, sw_v0, sa_v0, w_v0, a_v0, sem_w0, sem_a0, sem_o0),
            (idx_v1, sw_v1, sa_v1, w_v1, a_v1, sem_w1, sem_a1, sem_o1))

    def start_gathers(ci, b):
        idx_v, sw_v, sa_v, w_v, a_v, sem_w, sem_a, _ = bufs[b]
        tok = pl.multiple_of(base + ci * C, C)
        pltpu.sync_copy(idx_hbm.at[pl.ds(tok, C)], idx_v)

        def perm(g):
            off = pl.multiple_of(g * 16, 16)
            v = idx_v[pl.ds(off, 16)]
            rw = v & 8191
            sw_v[pl.ds(off, 16)] = (v - rw) + ((rw & 2047) << 2) + (rw >> 11)
            ra = v & 32767
            sa_v[pl.ds(off, 16)] = (v - ra) + ((ra & 2047) << 4) + (ra >> 11)

        plsc.parallel_loop(0, C // 16, 1, unroll=4, carry=None)(perm)
        pltpu.make_async_copy(weight_hbm.at[sw_v], w_v, sem_w).start()
        pltpu.make_async_copy(lora_a_hbm.at[sa_v], a_v, sem_a).start()

    def wait_gathers(b):
        idx_v, sw_v, sa_v, w_v, a_v, sem_w, sem_a, _ = bufs[b]
        pltpu.make_async_copy(weight_hbm.at[sw_v], w_v, sem_w).wait()
        pltpu.make_async_copy(lora_a_hbm.at[sa_v], a_v, sem_a).wait()

    def start_write(ci, b):
        w_v, sem_o = bufs[b][3], bufs[b][7]
        row = pl.multiple_of(rbase + ci * CB, CB)
        for j in range(CB):
            pltpu.make_async_copy(
                w_v.at[pl.ds(j * 50, 50)], out_hbm.at[row + j], sem_o).start()

    def wait_write(b):
        w_v, sem_o = bufs[b][3], bufs[b][7]
        for j in range(CB):
            pltpu.make_async_copy(
                w_v.at[pl.ds(0, 50)], out_hbm.at[0], sem_o).wait()

    def compute(b):
        w_v, a_v = bufs[b][3], bufs[b][4]

        def pair_body(p):
            t0 = p * 2
            av = plsc.load_gather(a_v, [t0 + row_step, col_idx])
            acc00 = w_v[t0, pl.ds(0, 16)]
            acc01 = w_v[t0, pl.ds(16, 16)]
            acc10 = w_v[t0 + 1, pl.ds(0, 16)]
            acc11 = w_v[t0 + 1, pl.ds(16, 16)]
            for r in range(RANK):
                b0 = lb_v[r, pl.ds(0, 16)]
                b1 = lb_v[r, pl.ds(16, 16)]
                s0 = av[r] * SCALE
                s1 = av[r + RANK] * SCALE
                acc00 = acc00 + s0 * b0
                acc01 = acc01 + s0 * b1
                acc10 = acc10 + s1 * b0
                acc11 = acc11 + s1 * b1
            w_v[t0, pl.ds(0, 16)] = acc00
            w_v[t0, pl.ds(16, 16)] = acc01
            w_v[t0 + 1, pl.ds(0, 16)] = acc10
            w_v[t0 + 1, pl.ds(16, 16)] = acc11

        plsc.parallel_loop(0, C // 2, 1, unroll=4, carry=None)(pair_body)

    start_gathers(0, 0)

    def outer(cj, _):
        for b in (0, 1):
            ci = cj * 2 + b
            wait_gathers(b)
            nb = 1 - b
            @pl.when(ci >= 1)
            def _():
                wait_write(nb)

            @pl.when(ci + 1 < N_CHUNK)
            def _():
                start_gathers(ci + 1, nb)

            compute(b)
            start_write(ci, b)
        return 0

    lax.fori_loop(0, N_CHUNK // 2, outer, 0)
    wait_write(1)


def _detile_body(x_ref, y_ref):
    x = x_ref[...]
    rows, vb = x.shape
    npc = 128 // rows
    w = vb // npc
    pieces = [jnp.transpose(x[:, j * w:(j + 1) * w]) for j in range(npc)]
    y_ref[...] = jnp.concatenate(pieces, axis=1)


def _detile(xt, vb):
    import math
    rows, v = xt.shape
    npc = 128 // rows
    w = vb // npc
    grid = math.ceil(v / vb)
    return pl.pallas_call(
        _detile_body,
        grid=(grid,),
        in_specs=[pl.BlockSpec((rows, vb), lambda i: (0, i))],
        out_specs=pl.BlockSpec((w, 128), lambda i: (i, 0)),
        out_shape=jax.ShapeDtypeStruct((grid * w, 128), jnp.float32),
    )(xt)


@jax.jit
def _lora_embed(weight, idx, lora_a, lora_b):
    mesh = plsc.VectorSubcoreMesh(core_axis_name="c", subcore_axis_name="s")
    fn = pl.kernel(
        _body,
        out_type=jax.ShapeDtypeStruct((16384, 50, DIM), jnp.float32),
        mesh=mesh,
        compiler_params=pltpu.CompilerParams(
            needs_layout_passes=False, use_tc_tiling_on_sc=False),
        scratch_types=[
            pltpu.VMEM((C,), jnp.int32),
            pltpu.VMEM((C,), jnp.int32),
            pltpu.VMEM((C,), jnp.int32),
            pltpu.VMEM((C,), jnp.int32),
            pltpu.VMEM((C,), jnp.int32),
            pltpu.VMEM((C,), jnp.int32),
            pltpu.VMEM((C, DIM), jnp.float32),
            pltpu.VMEM((C, DIM), jnp.float32),
            pltpu.VMEM((C, RANK), jnp.float32),
            pltpu.VMEM((C, RANK), jnp.float32),
            pltpu.VMEM((RANK, DIM), jnp.float32),
            pltpu.SemaphoreType.DMA,
            pltpu.SemaphoreType.DMA,
            pltpu.SemaphoreType.DMA,
            pltpu.SemaphoreType.DMA,
            pltpu.SemaphoreType.DMA,
            pltpu.SemaphoreType.DMA,
        ],
    )
    return fn(weight, idx, lora_a, lora_b)


def kernel(x, weight, lora_A, lora_B):
    idx = x.reshape(-1).astype(jnp.int32)
    wperm = _detile(weight.T, 8192)
    w_rm = wperm.reshape(-1, DIM)
    aperm = _detile(lora_A.T, 32768)
    a_rm = aperm.reshape(-1, RANK)
    return _lora_embed(w_rm, idx, a_rm, lora_B)


# speedup vs baseline: 3.7266x; 2.1006x over previous
import math

import jax
import jax.numpy as jnp
from jax import lax
from jax.experimental import pallas as pl
from jax.experimental.pallas import tpu as pltpu
from jax.experimental.pallas import tpu_sc as plsc

VOCAB = 1000000
DIM = 32
RANK = 8
SCALE = 1.0

NUM_CORES = 2
NUM_SUBCORES = 16
NW = NUM_CORES * NUM_SUBCORES
NB = 16384
NL = 50
N_TOK = NB * NL
TOK_PER_W = N_TOK // NW
C = 800
N_CHUNK = TOK_PER_W // C

W_PIECE = 2048
W_VB = 4 * W_PIECE


def _merge_detile_body(wt_ref, at_ref, lb_ref, y_ref):
    m = wt_ref[...] + jax.lax.dot_general(
        lb_ref[...], at_ref[...],
        dimension_numbers=(((0,), (0,)), ((), ())),
        preferred_element_type=jnp.float32) * SCALE
    pieces = [jnp.transpose(m[:, j * W_PIECE:(j + 1) * W_PIECE])
              for j in range(4)]
    y_ref[...] = jnp.concatenate(pieces, axis=1)


def _merge_detile(wt, at, lb):
    grid = math.ceil(VOCAB / W_VB)
    return pl.pallas_call(
        _merge_detile_body,
        grid=(grid,),
        in_specs=[
            pl.BlockSpec((DIM, W_VB), lambda i: (0, i)),
            pl.BlockSpec((RANK, W_VB), lambda i: (0, i)),
            pl.BlockSpec((RANK, DIM), lambda i: (0, 0)),
        ],
        out_specs=pl.BlockSpec((W_PIECE, 128), lambda i: (i, 0)),
        out_shape=jax.ShapeDtypeStruct((grid * W_PIECE, 128), jnp.float32),
    )(wt, at, lb)


def _retile_body(x_ref, y_ref):
    x = x_ref[...]
    pieces = [jnp.transpose(x[:, j * DIM:(j + 1) * DIM]) for j in range(4)]
    y_ref[...] = jnp.concatenate(pieces, axis=1)


def _retile(rows128):
    return pl.pallas_call(
        _retile_body,
        grid=(NL,),
        in_specs=[pl.BlockSpec((NB // 4, 128), lambda l: (l, 0))],
        out_specs=pl.BlockSpec((DIM, NB), lambda l: (l, 0)),
        out_shape=jax.ShapeDtypeStruct((NL * DIM, NB), jnp.float32),
    )(rows128)


def _gather_body(table_hbm, idx_hbm, out_hbm,
                 idx_v0, idx_v1, sw_v0, sw_v1, so_v0, so_v1, w_v0, w_v1,
                 sem_w0, sem_w1, sem_o0, sem_o1):
    wid = lax.axis_index("s") * NUM_CORES + lax.axis_index("c")
    base = wid * TOK_PER_W

    lane = lax.iota(jnp.int32, 16)

    bufs = ((idx_v0, sw_v0, so_v0, w_v0, sem_w0, sem_o0),
            (idx_v1, sw_v1, so_v1, w_v1, sem_w1, sem_o1))

    def start_gathers(ci, b):
        idx_v, sw_v, so_v, w_v, sem_w, _ = bufs[b]
        tok = pl.multiple_of(base + ci * C, C)
        pltpu.sync_copy(idx_hbm.at[pl.ds(tok, C)], idx_v)

        def perm(g):
            off = pl.multiple_of(g * 16, 16)
            v = idx_v[pl.ds(off, 16)]
            rw = v & (W_VB - 1)
            sw_v[pl.ds(off, 16)] = (
                (v - rw) + ((rw & (W_PIECE - 1)) << 2) + (rw >> 11))
            t = tok + off + lane
            bb = t // NL
            l = t - bb * NL
            so_v[pl.ds(off, 16)] = (
                l * NB + ((bb & 4095) << 2) + (bb >> 12))

        plsc.parallel_loop(0, C // 16, 1, unroll=4, carry=None)(perm)
        pltpu.make_async_copy(table_hbm.at[sw_v], w_v, sem_w).start()

    def wait_gathers(b):
        _, sw_v, _, w_v, sem_w, _ = bufs[b]
        pltpu.make_async_copy(table_hbm.at[sw_v], w_v, sem_w).wait()

    def start_write(b):
        _, _, so_v, w_v, _, sem_o = bufs[b]
        pltpu.make_async_copy(w_v, out_hbm.at[so_v], sem_o).start()

    def wait_write(b):
        _, _, so_v, w_v, _, sem_o = bufs[b]
        pltpu.make_async_copy(w_v, out_hbm.at[so_v], sem_o).wait()

    start_gathers(0, 0)

    def outer(cj, _):
        for b in (0, 1):
            ci = cj * 2 + b
            wait_gathers(b)
            nb = 1 - b

            @pl.when(ci >= 1)
            def _():
                wait_write(nb)

            @pl.when(ci + 1 < N_CHUNK)
            def _():
                start_gathers(ci + 1, nb)

            start_write(b)
        return 0

    lax.fori_loop(0, N_CHUNK // 2, outer, 0)
    wait_write(1)


@jax.jit
def _lora_embed(table, idx):
    mesh = plsc.VectorSubcoreMesh(core_axis_name="c", subcore_axis_name="s")
    fn = pl.kernel(
        _gather_body,
        out_type=jax.ShapeDtypeStruct((N_TOK, DIM), jnp.float32),
        mesh=mesh,
        compiler_params=pltpu.CompilerParams(
            needs_layout_passes=False, use_tc_tiling_on_sc=False),
        scratch_types=[
            pltpu.VMEM((C,), jnp.int32),
            pltpu.VMEM((C,), jnp.int32),
            pltpu.VMEM((C,), jnp.int32),
            pltpu.VMEM((C,), jnp.int32),
            pltpu.VMEM((C,), jnp.int32),
            pltpu.VMEM((C,), jnp.int32),
            pltpu.VMEM((C, DIM), jnp.float32),
            pltpu.VMEM((C, DIM), jnp.float32),
            pltpu.SemaphoreType.DMA,
            pltpu.SemaphoreType.DMA,
            pltpu.SemaphoreType.DMA,
            pltpu.SemaphoreType.DMA,
        ],
    )
    return fn(table, idx)


def kernel(x, weight, lora_A, lora_B):
    idx = x.reshape(-1).astype(jnp.int32)
    merged = _merge_detile(weight.T, lora_A.T, lora_B)
    table = merged.reshape(-1, DIM)
    rows = _lora_embed(table, idx)
    y2 = _retile(rows.reshape(N_TOK // 4, 128))
    return jnp.transpose(y2.reshape(NL, DIM, NB), (2, 0, 1))
